# baseline (speedup 1.0000x reference)
import functools

import jax
import jax.numpy as jnp
from jax import lax
from jax.experimental import pallas as pl
from jax.experimental.pallas import tpu as pltpu
from jax.experimental.pallas import tpu_sc as plsc

_NC = 2
_NS = 16
_NW = _NC * _NS
_B = 80
_DEGW = 16


def _sc_mesh():
    return plsc.VectorSubcoreMesh(core_axis_name="c", subcore_axis_name="s")


def _pad_rows(n, align=8):
    step = align * _NS
    return ((n + step - 1) // step) * step


_K = 5


def _make_deg_kernel(n_pad, e):
    rpt = n_pad // _NS
    ept = e // _NW
    nsuper = ept // (_B * _K)
    ka, kb = 3, 2
    assert ka + kb == _K

    @functools.partial(
        pl.kernel,
        out_type=jax.ShapeDtypeStruct((_NC, n_pad, _DEGW), jnp.float32),
        mesh=_sc_mesh(),
        scratch_types=(
            [pltpu.VMEM((_B,), jnp.int32) for _ in range(_K)]
            + [
                pltpu.VMEM((_B, _DEGW), jnp.float32),
                pltpu.VMEM_SHARED((n_pad, _DEGW), jnp.float32),
                pltpu.SemaphoreType.DMA,
                pltpu.SemaphoreType.DMA,
                pltpu.SemaphoreType.DMA,
                pltpu.SemaphoreType.DMA,
            ]
        ),
        compiler_params=pltpu.CompilerParams(use_tc_tiling_on_sc=False),
    )
    def deg_kernel(dst_hbm, zeros_hbm, ones_hbm, out_hbm, *scr):
        didx = scr[:_K]
        didx_a, didx_b = didx[:ka], didx[ka:]
        ones_v, deg_sp, isem_a, isem_b, ssem_a, ssem_b = scr[_K:]
        c = lax.axis_index("c")
        s = lax.axis_index("s")
        wid = c * _NS + s
        r0 = s * rpt
        pltpu.sync_copy(zeros_hbm, deg_sp.at[pl.ds(r0, rpt)])
        pltpu.sync_copy(ones_hbm, ones_v)
        plsc.subcore_barrier()
        e0 = wid * ept

        def fire_idx(chunk0, didxs, isem):
            for i, dref in enumerate(didxs):
                pltpu.async_copy(
                    dst_hbm.at[pl.ds(e0 + (chunk0 + i) * _B, _B)], dref, isem)

        def drain_idx(didxs, isem):
            for dref in didxs:
                pltpu.make_async_copy(dst_hbm.at[pl.ds(0, _B)], dref, isem).wait()

        def fire_scatters(didxs, ssem):
            for dref in didxs:
                pltpu.async_copy(ones_v, deg_sp.at[dref], ssem, add=True)

        def drain_scatters(didxs, ssem):
            for _ in didxs:
                pltpu.make_async_copy(ones_hbm, ones_v, ssem).wait()

        fire_idx(0, didx_a, isem_a)

        def body(g, carry):
            cbase = g * _K

            @pl.when(g > 0)
            def _():
                drain_scatters(didx_b, ssem_b)

            fire_idx(cbase + ka, didx_b, isem_b)
            drain_idx(didx_a, isem_a)
            fire_scatters(didx_a, ssem_a)
            drain_scatters(didx_a, ssem_a)

            @pl.when(g < nsuper - 1)
            def _():
                fire_idx(cbase + _K, didx_a, isem_a)

            drain_idx(didx_b, isem_b)
            fire_scatters(didx_b, ssem_b)
            return carry

        lax.fori_loop(0, nsuper, body, 0)
        drain_scatters(didx_b, ssem_b)
        plsc.subcore_barrier()
        pltpu.sync_copy(deg_sp.at[pl.ds(r0, rpt)], out_hbm.at[c].at[pl.ds(r0, rpt)])

    return deg_kernel


def _make_scatter_kernel(n_pad, e, d):
    rpt = n_pad // _NS
    ept = e // _NW
    sets = ((0, 56), (56, 48), (104, 48), (152, 48))
    grp = sum(sz for _, sz in sets)
    ngroup = ept // grp

    @functools.partial(
        pl.kernel,
        out_type=jax.ShapeDtypeStruct((_NC, n_pad, d), jnp.float32),
        mesh=_sc_mesh(),
        scratch_types=(
            [pltpu.VMEM((ept,), jnp.int32)]
            + [pltpu.VMEM((sz,), jnp.int32) for _, sz in sets]
            + [pltpu.VMEM((sz, d), jnp.float32) for _, sz in sets]
            + [pltpu.VMEM_SHARED((n_pad, d), jnp.float32)]
            + [pltpu.SemaphoreType.DMA] * (2 * len(sets))
        ),
    )
    def scatter_kernel(src_hbm, dst_hbm, g_hbm, zeros_hbm, out_hbm, *scr):
        ns = len(sets)
        src_big = scr[0]
        didx = scr[1:1 + ns]
        rows = scr[1 + ns:1 + 2 * ns]
        acc_sp = scr[1 + 2 * ns]
        gsem = scr[2 + 2 * ns:2 + 2 * ns + ns]
        ssem = scr[2 + 2 * ns + ns:]
        c = lax.axis_index("c")
        s = lax.axis_index("s")
        wid = c * _NS + s
        r0 = s * rpt
        e0 = wid * ept
        pltpu.sync_copy(zeros_hbm, acc_sp.at[pl.ds(r0, rpt)])
        pltpu.sync_copy(src_hbm.at[pl.ds(e0, ept)], src_big)
        plsc.subcore_barrier()

        def fire_g(i, gbase):
            off, sz = sets[i]
            pltpu.async_copy(dst_hbm.at[pl.ds(e0 + gbase + off, sz)],
                             didx[i], gsem[i])
            pltpu.async_copy(g_hbm.at[src_big.at[pl.ds(gbase + off, sz)]],
                             rows[i], gsem[i])

        def drain_g(i):
            _, sz = sets[i]
            pltpu.make_async_copy(g_hbm.at[pl.ds(0, sz)], rows[i],
                                  gsem[i]).wait()
            pltpu.make_async_copy(dst_hbm.at[pl.ds(0, sz)], didx[i],
                                  gsem[i]).wait()

        def fire_s(i):
            pltpu.async_copy(rows[i], acc_sp.at[didx[i]], ssem[i], add=True)

        def drain_s(i):
            _, sz = sets[i]
            pltpu.make_async_copy(g_hbm.at[pl.ds(0, sz)], rows[i],
                                  ssem[i]).wait()

        for i in range(ns - 1):
            fire_g(i, 0)

        def body(g, carry):
            gbase = g * grp
            for i in range(ns):
                drain_g(i)
                fire_s(i)
                j = (i - 1) % ns
                if i == 0:
                    @pl.when(g > 0)
                    def _():
                        drain_s(j)
                    fire_g(j, gbase)
                else:
                    drain_s(j)

                    @pl.when(g < ngroup - 1)
                    def _():
                        fire_g(j, gbase + grp)
            return carry

        lax.fori_loop(0, ngroup, body, 0)
        drain_s(ns - 1)
        plsc.subcore_barrier()
        pltpu.sync_copy(acc_sp.at[pl.ds(r0, rpt)], out_hbm.at[c].at[pl.ds(r0, rpt)])

    return scatter_kernel


def _deg_from_partials(dp_ref):
    return dp_ref[0, :, 0:1] + dp_ref[1, :, 0:1] + 1.0


def _tc_scale_body(x_ref, w_ref, dp_ref, g_ref):
    h = jnp.dot(x_ref[...], w_ref[...], preferred_element_type=jnp.float32)
    g_ref[...] = h * lax.rsqrt(_deg_from_partials(dp_ref))


def _tc_final_body(s_ref, g_ref, dp_ref, b_ref, o_ref):
    dinv = lax.rsqrt(_deg_from_partials(dp_ref))
    agg = s_ref[0] + s_ref[1] + g_ref[...]
    o_ref[...] = jnp.maximum(dinv * agg + b_ref[...], 0.0)


def kernel(x, edge_index, W, b):
    n, d = x.shape
    e = edge_index.shape[1]
    src = edge_index[0]
    dst = edge_index[1]
    n_pad = _pad_rows(n)
    rpt = n_pad // _NS
    zeros16 = jnp.zeros((rpt, _DEGW), jnp.float32)
    ones16 = jnp.ones((_B, _DEGW), jnp.float32)
    zeros_d = jnp.zeros((rpt, d), jnp.float32)

    deg_partials = _make_deg_kernel(n_pad, e)(dst, zeros16, ones16)

    blk = 2000
    grid = n // blk
    g = pl.pallas_call(
        _tc_scale_body,
        out_shape=jax.ShapeDtypeStruct((n, d), jnp.float32),
        grid=(grid,),
        in_specs=[
            pl.BlockSpec((blk, d), lambda i: (i, 0)),
            pl.BlockSpec((d, d), lambda i: (0, 0)),
            pl.BlockSpec((_NC, blk, _DEGW), lambda i: (0, i, 0)),
        ],
        out_specs=pl.BlockSpec((blk, d), lambda i: (i, 0)),
    )(x, W, deg_partials)

    s_partials = _make_scatter_kernel(n_pad, e, d)(src, dst, g, zeros_d)

    out = pl.pallas_call(
        _tc_final_body,
        out_shape=jax.ShapeDtypeStruct((n, d), jnp.float32),
        grid=(grid,),
        in_specs=[
            pl.BlockSpec((_NC, blk, d), lambda i: (0, i, 0)),
            pl.BlockSpec((blk, d), lambda i: (i, 0)),
            pl.BlockSpec((_NC, blk, _DEGW), lambda i: (0, i, 0)),
            pl.BlockSpec((1, d), lambda i: (0, 0)),
        ],
        out_specs=pl.BlockSpec((blk, d), lambda i: (i, 0)),
    )(s_partials, g, deg_partials, b.reshape(1, d))

    return out


# speedup vs baseline: 1.0298x; 1.0298x over previous
import functools

import jax
import jax.numpy as jnp
from jax import lax
from jax.experimental import pallas as pl
from jax.experimental.pallas import tpu as pltpu
from jax.experimental.pallas import tpu_sc as plsc

_NC = 2
_NS = 16
_NW = _NC * _NS
_B = 80
_DEGW = 16


def _sc_mesh():
    return plsc.VectorSubcoreMesh(core_axis_name="c", subcore_axis_name="s")


def _pad_rows(n, align=8):
    step = align * _NS
    return ((n + step - 1) // step) * step


_K = 5


def _make_deg_kernel(n_pad, e):
    rpt = n_pad // _NS
    ept = e // _NW
    nsuper = ept // (_B * _K)
    ka, kb = 3, 2
    assert ka + kb == _K

    @functools.partial(
        pl.kernel,
        out_type=jax.ShapeDtypeStruct((_NC, n_pad, _DEGW), jnp.float32),
        mesh=_sc_mesh(),
        scratch_types=(
            [pltpu.VMEM((_B,), jnp.int32) for _ in range(_K)]
            + [
                pltpu.VMEM((_B, _DEGW), jnp.float32),
                pltpu.VMEM_SHARED((n_pad, _DEGW), jnp.float32),
                pltpu.SemaphoreType.DMA,
                pltpu.SemaphoreType.DMA,
                pltpu.SemaphoreType.DMA,
                pltpu.SemaphoreType.DMA,
            ]
        ),
        compiler_params=pltpu.CompilerParams(use_tc_tiling_on_sc=False),
    )
    def deg_kernel(dst_hbm, zeros_hbm, ones_hbm, out_hbm, *scr):
        didx = scr[:_K]
        didx_a, didx_b = didx[:ka], didx[ka:]
        ones_v, deg_sp, isem_a, isem_b, ssem_a, ssem_b = scr[_K:]
        c = lax.axis_index("c")
        s = lax.axis_index("s")
        wid = c * _NS + s
        r0 = s * rpt
        pltpu.sync_copy(zeros_hbm, deg_sp.at[pl.ds(r0, rpt)])
        pltpu.sync_copy(ones_hbm, ones_v)
        plsc.subcore_barrier()
        e0 = wid * ept

        def fire_idx(chunk0, didxs, isem):
            for i, dref in enumerate(didxs):
                pltpu.async_copy(
                    dst_hbm.at[pl.ds(e0 + (chunk0 + i) * _B, _B)], dref, isem)

        def drain_idx(didxs, isem):
            for dref in didxs:
                pltpu.make_async_copy(dst_hbm.at[pl.ds(0, _B)], dref, isem).wait()

        def fire_scatters(didxs, ssem):
            for dref in didxs:
                pltpu.async_copy(ones_v, deg_sp.at[dref], ssem, add=True)

        def drain_scatters(didxs, ssem):
            for _ in didxs:
                pltpu.make_async_copy(ones_hbm, ones_v, ssem).wait()

        fire_idx(0, didx_a, isem_a)

        def body(g, carry):
            cbase = g * _K

            @pl.when(g > 0)
            def _():
                drain_scatters(didx_b, ssem_b)

            fire_idx(cbase + ka, didx_b, isem_b)
            drain_idx(didx_a, isem_a)
            fire_scatters(didx_a, ssem_a)
            drain_scatters(didx_a, ssem_a)

            @pl.when(g < nsuper - 1)
            def _():
                fire_idx(cbase + _K, didx_a, isem_a)

            drain_idx(didx_b, isem_b)
            fire_scatters(didx_b, ssem_b)
            return carry

        lax.fori_loop(0, nsuper, body, 0)
        drain_scatters(didx_b, ssem_b)
        plsc.subcore_barrier()
        pltpu.sync_copy(deg_sp.at[pl.ds(r0, rpt)], out_hbm.at[c].at[pl.ds(r0, rpt)])

    return deg_kernel


def _make_scatter_kernel(n_pad, e, d):
    rpt = n_pad // _NS
    ept = e // _NW
    sets = ((0, 72), (72, 64), (136, 64))
    grp = sum(sz for _, sz in sets)
    ngroup = ept // grp

    @functools.partial(
        pl.kernel,
        out_type=jax.ShapeDtypeStruct((_NC, n_pad, d), jnp.float32),
        mesh=_sc_mesh(),
        scratch_types=(
            [pltpu.VMEM((ept,), jnp.int32)]
            + [pltpu.VMEM((sz,), jnp.int32) for _, sz in sets]
            + [pltpu.VMEM((sz, d), jnp.float32) for _, sz in sets]
            + [pltpu.VMEM_SHARED((n_pad, d), jnp.float32)]
            + [pltpu.SemaphoreType.DMA] * (2 * len(sets))
        ),
    )
    def scatter_kernel(src_hbm, dst_hbm, g_hbm, zeros_hbm, out_hbm, *scr):
        ns = len(sets)
        src_big = scr[0]
        didx = scr[1:1 + ns]
        rows = scr[1 + ns:1 + 2 * ns]
        acc_sp = scr[1 + 2 * ns]
        gsem = scr[2 + 2 * ns:2 + 2 * ns + ns]
        ssem = scr[2 + 2 * ns + ns:]
        c = lax.axis_index("c")
        s = lax.axis_index("s")
        wid = c * _NS + s
        r0 = s * rpt
        e0 = wid * ept
        pltpu.sync_copy(zeros_hbm, acc_sp.at[pl.ds(r0, rpt)])
        pltpu.sync_copy(src_hbm.at[pl.ds(e0, ept)], src_big)
        plsc.subcore_barrier()

        def fire_g(i, gbase):
            off, sz = sets[i]
            pltpu.async_copy(dst_hbm.at[pl.ds(e0 + gbase + off, sz)],
                             didx[i], gsem[i])
            pltpu.async_copy(g_hbm.at[src_big.at[pl.ds(gbase + off, sz)]],
                             rows[i], gsem[i])

        def drain_g(i):
            _, sz = sets[i]
            pltpu.make_async_copy(g_hbm.at[pl.ds(0, sz)], rows[i],
                                  gsem[i]).wait()
            pltpu.make_async_copy(dst_hbm.at[pl.ds(0, sz)], didx[i],
                                  gsem[i]).wait()

        def fire_s(i):
            pltpu.async_copy(rows[i], acc_sp.at[didx[i]], ssem[i], add=True)

        def drain_s(i):
            _, sz = sets[i]
            pltpu.make_async_copy(g_hbm.at[pl.ds(0, sz)], rows[i],
                                  ssem[i]).wait()

        for i in range(ns - 1):
            fire_g(i, 0)

        def body(g, carry):
            gbase = g * grp
            for i in range(ns):
                drain_g(i)
                fire_s(i)
                j = (i - 1) % ns
                if i == 0:
                    @pl.when(g > 0)
                    def _():
                        drain_s(j)
                    fire_g(j, gbase)
                else:
                    drain_s(j)

                    @pl.when(g < ngroup - 1)
                    def _():
                        fire_g(j, gbase + grp)
            return carry

        lax.fori_loop(0, ngroup, body, 0)
        drain_s(ns - 1)
        plsc.subcore_barrier()
        pltpu.sync_copy(acc_sp.at[pl.ds(r0, rpt)], out_hbm.at[c].at[pl.ds(r0, rpt)])

    return scatter_kernel


def _deg_from_partials(dp_ref):
    return dp_ref[0, :, 0:1] + dp_ref[1, :, 0:1] + 1.0


def _tc_scale_body(x_ref, w_ref, dp_ref, g_ref):
    h = jnp.dot(x_ref[...], w_ref[...], preferred_element_type=jnp.float32)
    g_ref[...] = h * lax.rsqrt(_deg_from_partials(dp_ref))


def _tc_final_body(s_ref, g_ref, dp_ref, b_ref, o_ref):
    dinv = lax.rsqrt(_deg_from_partials(dp_ref))
    agg = s_ref[0] + s_ref[1] + g_ref[...]
    o_ref[...] = jnp.maximum(dinv * agg + b_ref[...], 0.0)


def kernel(x, edge_index, W, b):
    n, d = x.shape
    e = edge_index.shape[1]
    src = edge_index[0]
    dst = edge_index[1]
    n_pad = _pad_rows(n)
    rpt = n_pad // _NS
    zeros16 = jnp.zeros((rpt, _DEGW), jnp.float32)
    ones16 = jnp.ones((_B, _DEGW), jnp.float32)
    zeros_d = jnp.zeros((rpt, d), jnp.float32)

    deg_partials = _make_deg_kernel(n_pad, e)(dst, zeros16, ones16)

    blk = 2000
    grid = n // blk
    g = pl.pallas_call(
        _tc_scale_body,
        out_shape=jax.ShapeDtypeStruct((n, d), jnp.float32),
        grid=(grid,),
        in_specs=[
            pl.BlockSpec((blk, d), lambda i: (i, 0)),
            pl.BlockSpec((d, d), lambda i: (0, 0)),
            pl.BlockSpec((_NC, blk, _DEGW), lambda i: (0, i, 0)),
        ],
        out_specs=pl.BlockSpec((blk, d), lambda i: (i, 0)),
    )(x, W, deg_partials)

    s_partials = _make_scatter_kernel(n_pad, e, d)(src, dst, g, zeros_d)

    out = pl.pallas_call(
        _tc_final_body,
        out_shape=jax.ShapeDtypeStruct((n, d), jnp.float32),
        grid=(grid,),
        in_specs=[
            pl.BlockSpec((_NC, blk, d), lambda i: (0, i, 0)),
            pl.BlockSpec((blk, d), lambda i: (i, 0)),
            pl.BlockSpec((_NC, blk, _DEGW), lambda i: (0, i, 0)),
            pl.BlockSpec((1, d), lambda i: (0, 0)),
        ],
        out_specs=pl.BlockSpec((blk, d), lambda i: (i, 0)),
    )(s_partials, g, deg_partials, b.reshape(1, d))

    return out
